# Initial kernel scaffold; baseline (speedup 1.0000x reference)
import jax
import jax.numpy as jnp
from jax.experimental import pallas as pl


def kernel(embedding, selection_idx, selection_prob, W, b):
    raise NotImplementedError("write your pallas kernel here")


# speedup vs baseline: 2.0013x; 2.0013x over previous
import functools

import jax
import jax.numpy as jnp
from jax import lax
from jax.experimental import pallas as pl
from jax.experimental.pallas import tpu as pltpu
from jax.experimental.pallas import tpu_sc as plsc


def _matmul_body(emb_ref, w_ref, b_ref, out_ref):
    out_ref[...] = (
        jnp.dot(emb_ref[...], w_ref[...], preferred_element_type=jnp.float32)
        + b_ref[...]
    )


@functools.partial(jax.jit, static_argnames=("bs", "d", "he", "t"))
def _all_bank_projection(emb2d, w2d, b2d, *, bs, d, he, t):
    grid = (bs // t,)
    return pl.pallas_call(
        _matmul_body,
        grid=grid,
        in_specs=[
            pl.BlockSpec((t, d), lambda i: (i, 0)),
            pl.BlockSpec((d, he), lambda i: (0, 0)),
            pl.BlockSpec((1, he), lambda i: (0, 0)),
        ],
        out_specs=pl.BlockSpec((t, he), lambda i: (i, 0)),
        out_shape=jax.ShapeDtypeStruct((bs, he), jnp.float32),
    )(emb2d, w2d, b2d)


def _make_sc_gather(rows, e, nc, ns):
    nw = nc * ns
    per_w = rows // nw
    chunk = 128
    n_chunks = per_w // chunk
    mesh = plsc.VectorSubcoreMesh(core_axis_name="c", subcore_axis_name="s")

    @functools.partial(
        pl.kernel,
        out_type=jax.ShapeDtypeStruct((rows, e), jnp.float32),
        mesh=mesh,
        compiler_params=pltpu.CompilerParams(use_tc_tiling_on_sc=False),
        scratch_types=[
            pltpu.VMEM((chunk,), jnp.int32),
            pltpu.VMEM((chunk, e), jnp.float32),
            pltpu.SemaphoreType.DMA,
        ],
    )
    def gather_kernel(table_hbm, idx_hbm, out_hbm, idx_v, rows_v, sem):
        wid = lax.axis_index("s") * nc + lax.axis_index("c")
        base = wid * per_w

        def do_chunk(c, carry):
            start = base + c * chunk
            pltpu.sync_copy(idx_hbm.at[pl.ds(start, chunk)], idx_v)
            pltpu.async_copy(table_hbm.at[idx_v], rows_v, sem).wait()
            pltpu.sync_copy(rows_v, out_hbm.at[pl.ds(start, chunk)])
            return carry

        lax.fori_loop(0, n_chunks, do_chunk, 0)

    return gather_kernel


def kernel(embedding, selection_idx, selection_prob, W, b):
    del selection_prob
    bb, s, d = embedding.shape
    h, _, e = W.shape
    k = selection_idx.shape[-1]
    bs = bb * s

    emb2d = embedding.reshape(bs, d)
    w2d = jnp.transpose(W, (1, 0, 2)).reshape(d, h * e)
    b2d = b.reshape(1, h * e)
    all_out = _all_bank_projection(emb2d, w2d, b2d, bs=bs, d=d, he=h * e, t=512)

    table = all_out.reshape(bs * h, e)
    sel_flat = selection_idx.astype(jnp.int32).reshape(bs * k)
    tok_ids = jnp.arange(bs * k, dtype=jnp.int32) // k
    flat_idx = tok_ids * h + sel_flat

    info = plsc.get_sparse_core_info()
    gathered = _make_sc_gather(bs * k, e, info.num_cores, info.num_subcores)(
        table, flat_idx
    )
    return gathered.reshape(bb, s, k, e)


# speedup vs baseline: 2.0212x; 1.0099x over previous
import functools

import jax
import jax.numpy as jnp
from jax import lax
from jax.experimental import pallas as pl
from jax.experimental.pallas import tpu as pltpu
from jax.experimental.pallas import tpu_sc as plsc


def _matmul_body(emb_ref, w_ref, b_ref, out_ref):
    out_ref[...] = (
        jnp.dot(emb_ref[...], w_ref[...], preferred_element_type=jnp.float32)
        + b_ref[...]
    )


@functools.partial(jax.jit, static_argnames=("bs", "d", "he", "t"))
def _all_bank_projection(emb2d, w2d, b2d, *, bs, d, he, t):
    grid = (bs // t,)
    return pl.pallas_call(
        _matmul_body,
        grid=grid,
        in_specs=[
            pl.BlockSpec((t, d), lambda i: (i, 0)),
            pl.BlockSpec((d, he), lambda i: (0, 0)),
            pl.BlockSpec((1, he), lambda i: (0, 0)),
        ],
        out_specs=pl.BlockSpec((t, he), lambda i: (i, 0)),
        out_shape=jax.ShapeDtypeStruct((bs, he), jnp.float32),
    )(emb2d, w2d, b2d)


def _make_sc_select(bs, he, e, h, k, nc, ns):
    nw = nc * ns
    toks_per_w = bs // nw
    tg = 16
    n_groups = toks_per_w // tg
    rows_g = tg * k
    mesh = plsc.VectorSubcoreMesh(core_axis_name="c", subcore_axis_name="s")

    @functools.partial(
        pl.kernel,
        out_type=jax.ShapeDtypeStruct((bs * k, e), jnp.float32),
        mesh=mesh,
        scratch_types=[
            pltpu.VMEM((tg, he), jnp.float32),
            pltpu.VMEM((rows_g, e), jnp.float32),
            pltpu.VMEM((rows_g,), jnp.int32),
        ],
    )
    def select_kernel(table_hbm, idx_hbm, out_hbm, tbuf, obuf, idx_s):
        wid = lax.axis_index("s") * nc + lax.axis_index("c")
        tok0 = wid * toks_per_w

        def do_group(g, carry):
            tstart = tok0 + g * tg
            rstart = tstart * k
            pltpu.sync_copy(table_hbm.at[pl.ds(tstart, tg)], tbuf)
            pltpu.sync_copy(idx_hbm.at[pl.ds(rstart, rows_g)], idx_s)
            for p in range(rows_g // 16):
                sv = idx_s[pl.ds(p * 16, 16)]
                for j in range(16):
                    row = p * 16 + j
                    tok = row // k
                    off = sv[j] * e
                    for i in range(0, e, 16):
                        obuf[row, pl.ds(i, 16)] = tbuf[tok, pl.ds(off + i, 16)]
            pltpu.sync_copy(obuf, out_hbm.at[pl.ds(rstart, rows_g)])
            return carry

        lax.fori_loop(0, n_groups, do_group, 0)

    return select_kernel


def kernel(embedding, selection_idx, selection_prob, W, b):
    del selection_prob
    bb, s, d = embedding.shape
    h, _, e = W.shape
    k = selection_idx.shape[-1]
    bs = bb * s

    emb2d = embedding.reshape(bs, d)
    w2d = jnp.transpose(W, (1, 0, 2)).reshape(d, h * e)
    b2d = b.reshape(1, h * e)
    all_out = _all_bank_projection(emb2d, w2d, b2d, bs=bs, d=d, he=h * e, t=512)

    sel_flat = selection_idx.astype(jnp.int32).reshape(bs * k)

    info = plsc.get_sparse_core_info()
    gathered = _make_sc_select(bs, h * e, e, h, k, info.num_cores, info.num_subcores)(
        all_out, sel_flat
    )
    return gathered.reshape(bb, s, k, e)
